# baseline (speedup 1.0000x reference)
import jax
import jax.numpy as jnp
from jax import lax
from jax.experimental import pallas as pl
from jax.experimental.pallas import tpu as pltpu
from jax.experimental.pallas import tpu_sc as plsc

N_NODES = 10000
N_EDGES = 320000
D_FEAT = 128
D_EDGE = 16
U_DIM = 16
B_GRAPHS = 64
K = 64

NC = 2
NS = 16
NW = NC * NS
E_PER_W = N_EDGES // NW
E_BLK = 1024
E_NBLK = 10
E_TAIL_ROWS = E_PER_W - (E_NBLK - 1) * E_BLK
N_CHUNKS_FULL = N_NODES // 128
N_TAIL = N_NODES - N_CHUNKS_FULL * 128
UNROLL = 4


def _sc_body(x_hbm, ei_hbm, ea_hbm, batch_hbm, pe_hbm, pn_hbm,
             col_v, batch_v, seg_v, rows0_v, rows1_v, xr0_v, xr1_v,
             eacc_v, nacc_v,
             sem_misc, sem_in0, sem_in1, sem_x0, sem_x1):
    c = lax.axis_index("c")
    s = lax.axis_index("s")
    wid = s * NC + c
    ebase = wid * E_PER_W
    rows = (rows0_v, rows1_v)
    sem_in = (sem_in0, sem_in1)

    def start_load(blk):
        buf = rows[blk % 2]
        if blk < E_NBLK - 1:
            return pltpu.async_copy(
                ea_hbm.at[pl.ds(ebase + blk * E_BLK, E_BLK), :],
                buf, sem_in[blk % 2])
        return pltpu.async_copy(
            ea_hbm.at[pl.ds(ebase + blk * E_BLK, E_TAIL_ROWS), :],
            buf.at[pl.ds(0, E_TAIL_ROWS), :], sem_in[blk % 2])

    d_batch = pltpu.async_copy(batch_hbm, batch_v, sem_misc)
    d_col = pltpu.async_copy(
        ei_hbm.at[pl.ds(N_EDGES + ebase, E_PER_W)], col_v, sem_misc)
    d_in0 = start_load(0)
    d_in1 = start_load(1)
    d_x0 = pltpu.async_copy(
        x_hbm.at[pl.ds(wid * 128, 128), :], xr0_v, sem_x0)
    d_x1 = pltpu.async_copy(
        x_hbm.at[pl.ds((wid + NW) * 128, 128), :], xr1_v, sem_x1)

    def zrow(r, carry):
        eacc_v[r, pl.ds(0, 16)] = jnp.zeros((16,), jnp.float32)
        for k in range(D_FEAT // 16):
            nacc_v[r, pl.ds(k * 16, 16)] = jnp.zeros((16,), jnp.float32)
        return carry
    lax.fori_loop(0, B_GRAPHS, zrow, 0)

    d_batch.wait()
    d_col.wait()

    def seg_i(i, carry):
        col16 = col_v[pl.ds(i * 16, 16)]
        seg_v[pl.ds(i * 16, 16)] = plsc.load_gather(batch_v, [col16])
        return carry
    lax.fori_loop(0, E_PER_W // 16, seg_i, 0, unroll=UNROLL)

    in_desc = [d_in0, d_in1] + [None] * (E_NBLK - 2)
    for blk in range(E_NBLK):
        cur = blk % 2
        in_desc[blk].wait()
        nrows = E_BLK if blk < E_NBLK - 1 else E_TAIL_ROWS

        def eadd(i, carry, cur=cur, blk=blk):
            seg16 = seg_v[pl.ds(blk * E_BLK + i * 16, 16)]
            for k in range(16):
                plsc.addupdate(eacc_v.at[seg16[k]], rows[cur][i * 16 + k, :])
            return carry
        lax.fori_loop(0, nrows // 16, eadd, 0)
        if blk + 2 < E_NBLK:
            in_desc[blk + 2] = start_load(blk + 2)

    def nproc(q, buf, n):
        def nadd(i, carry):
            b16 = batch_v[pl.ds(q * 128 + i * 16, 16)]
            for j in range(16):
                for k in range(D_FEAT // 16):
                    plsc.addupdate(nacc_v.at[b16[j], pl.ds(k * 16, 16)],
                                   buf[i * 16 + j, pl.ds(k * 16, 16)])
            return carry
        lax.fori_loop(0, n // 16, nadd, 0)

    d_x0.wait()
    nproc(wid, xr0_v, 128)
    d_x1.wait()
    nproc(wid + NW, xr1_v, 128)

    @pl.when(wid < N_CHUNKS_FULL - 2 * NW)
    def _third():
        q = wid + 2 * NW
        pltpu.sync_copy(x_hbm.at[pl.ds(q * 128, 128), :], xr0_v)
        nproc(q, xr0_v, 128)

    @pl.when(wid == NW - 1)
    def _tail():
        base = N_CHUNKS_FULL * 128
        pltpu.sync_copy(x_hbm.at[pl.ds(base, N_TAIL), :],
                        xr1_v.at[pl.ds(0, N_TAIL), :])
        nproc(N_CHUNKS_FULL, xr1_v, N_TAIL)

    pltpu.sync_copy(eacc_v, pe_hbm.at[wid])
    pltpu.sync_copy(nacc_v, pn_hbm.at[wid])


def _sc_aggregate(x, edge_index, edge_attr, batch):
    mesh = plsc.VectorSubcoreMesh(core_axis_name="c", subcore_axis_name="s")
    f32 = jnp.float32
    kern = pl.kernel(
        _sc_body,
        out_type=(
            jax.ShapeDtypeStruct((NW, B_GRAPHS, D_EDGE), f32),
            jax.ShapeDtypeStruct((NW, B_GRAPHS, D_FEAT), f32),
        ),
        mesh=mesh,
        compiler_params=pltpu.CompilerParams(
            needs_layout_passes=False, use_tc_tiling_on_sc=False),
        scratch_types=[
            pltpu.VMEM((E_PER_W,), jnp.int32),
            pltpu.VMEM((N_NODES,), jnp.int32),
            pltpu.VMEM((E_PER_W,), jnp.int32),
            pltpu.VMEM((E_BLK, D_EDGE), f32),
            pltpu.VMEM((E_BLK, D_EDGE), f32),
            pltpu.VMEM((128, D_FEAT), f32),
            pltpu.VMEM((128, D_FEAT), f32),
            pltpu.VMEM((B_GRAPHS, D_EDGE), f32),
            pltpu.VMEM((B_GRAPHS, D_FEAT), f32),
            pltpu.SemaphoreType.DMA,
            pltpu.SemaphoreType.DMA,
            pltpu.SemaphoreType.DMA,
            pltpu.SemaphoreType.DMA,
            pltpu.SemaphoreType.DMA,
        ],
    )
    return kern(x, edge_index.reshape(-1), edge_attr, batch)


def _mlp_body(u_ref, pe_ref, pn_ref, w1_ref, b1_ref, w2_ref, b2_ref, o_ref):
    hi = jax.lax.Precision.HIGHEST
    agg_e = jnp.sum(pe_ref[...], axis=0)
    agg_n = jnp.sum(pn_ref[...], axis=0)
    w1 = w1_ref[...]
    dn = (((1,), (0,)), ((), ()))
    z = (lax.dot_general(u_ref[...], w1[:U_DIM, :], dn, precision=hi)
         + lax.dot_general(agg_e, w1[U_DIM:U_DIM + D_EDGE, :], dn, precision=hi)
         + lax.dot_general(agg_n, w1[U_DIM + D_EDGE:, :], dn, precision=hi)
         + b1_ref[...][None, :])
    h = z * jax.nn.sigmoid(z)
    z2 = lax.dot_general(h, w2_ref[...], dn, precision=hi) + b2_ref[...][None, :]
    o_ref[...] = z2 * jax.nn.sigmoid(z2)


def _tc_mlp(u, pe, pn, W1, b1, W2, b2):
    return pl.pallas_call(
        _mlp_body,
        out_shape=jax.ShapeDtypeStruct((B_GRAPHS, K), jnp.float32),
    )(u, pe, pn, W1, b1, W2, b2)


@jax.jit
def kernel(x, edge_index, edge_attr, u, batch, W1, b1, W2, b2):
    pe, pn = _sc_aggregate(x, edge_index, edge_attr, batch)
    return _tc_mlp(u, pe, pn, W1, b1, W2, b2)
